# baseline (speedup 1.0000x reference)
import functools

import jax
import jax.numpy as jnp
from jax.experimental import pallas as pl
from jax.experimental.pallas import tpu as pltpu

F, B, D, K = 8, 1024, 32, 8192
BT = 256


def _vq_body(x_ref, w_ref, out_ref):
    x = x_ref[0]
    w = w_ref[0]
    xw = jax.lax.dot_general(
        x, w, (((1,), (0,)), ((), ())), preferred_element_type=jnp.float32
    )
    x2 = jnp.sum(x * x, axis=1, keepdims=True)
    w2 = jnp.sum(w * w, axis=0, keepdims=True)
    dist = x2 - 2.0 * xw + w2
    dmin = jnp.min(dist, axis=1, keepdims=True)
    iota = jax.lax.broadcasted_iota(jnp.int32, dist.shape, 1)
    idx = jnp.min(jnp.where(dist == dmin, iota, K), axis=1, keepdims=True)
    onehot = (iota == idx).astype(jnp.float32)
    q = jax.lax.dot_general(
        onehot, w, (((1,), (1,)), ((), ())), preferred_element_type=jnp.float32
    )
    out_ref[0] = x + (q - x)


@functools.partial(jax.jit, static_argnames=("interpret",))
def kernel(inputs, W, interpret=False):
    grid = (F, B // BT)
    out = pl.pallas_call(
        _vq_body,
        grid=grid,
        in_specs=[
            pl.BlockSpec((1, BT, D), lambda f, b: (f, b, 0)),
            pl.BlockSpec((1, D, K), lambda f, b: (f, 0, 0)),
        ],
        out_specs=pl.BlockSpec((1, BT, D), lambda f, b: (f, b, 0)),
        out_shape=jax.ShapeDtypeStruct((F, B, D), jnp.float32),
        compiler_params=pltpu.CompilerParams(
            dimension_semantics=("parallel", "parallel"),
        ),
        interpret=interpret,
    )(inputs, W)
    return out


# speedup vs baseline: 1.2752x; 1.2752x over previous
import functools

import jax
import jax.numpy as jnp
from jax import lax
from jax.experimental import pallas as pl
from jax.experimental.pallas import tpu as pltpu
from jax.experimental.pallas import tpu_sc as plsc

F, B, D, K = 8, 1024, 32, 8192
BT = 256
NB = B // BT

NC, NS = 2, 16
NW = NC * NS
BPW = (F * B) // NW
ICH = 128
NCH = BPW // ICH


def _argmin_body(x_ref, w_ref, idx_ref, wt_ref):
    f = pl.program_id(0)
    b = pl.program_id(1)
    x = x_ref[0]
    w = w_ref[0]
    xw = lax.dot_general(
        x, w, (((1,), (0,)), ((), ())), preferred_element_type=jnp.float32
    )
    x2 = jnp.sum(x * x, axis=1, keepdims=True)
    w2 = jnp.sum(w * w, axis=0, keepdims=True)
    dist = x2 - 2.0 * xw + w2
    dmin = jnp.min(dist, axis=1, keepdims=True)
    iota = lax.broadcasted_iota(jnp.int32, dist.shape, 1)
    ki = jnp.min(jnp.where(dist == dmin, iota, 2 * K), axis=1, keepdims=True)
    idx_ref[0] = ki + f * K

    @pl.when(b == 0)
    def _():
        wt_ref[0] = w.T


def _gather_body(table_hbm, idx_hbm, out_hbm, idx_v, rows_v, sem):
    wid = lax.axis_index("s") * NC + lax.axis_index("c")
    pltpu.sync_copy(idx_hbm.at[pl.ds(wid * NCH, NCH)], idx_v)
    copies = [
        pltpu.async_copy(
            table_hbm.at[idx_v.at[j]], rows_v.at[pl.ds(j * ICH, ICH)], sem
        )
        for j in range(NCH)
    ]
    for cp in copies:
        cp.wait()
    pltpu.sync_copy(rows_v, out_hbm.at[pl.ds(wid * BPW, BPW)])


@functools.partial(jax.jit, static_argnames=("interpret",))
def kernel(inputs, W, interpret=False):
    idx, wt = pl.pallas_call(
        _argmin_body,
        grid=(F, NB),
        in_specs=[
            pl.BlockSpec((1, BT, D), lambda f, b: (f, b, 0)),
            pl.BlockSpec((1, D, K), lambda f, b: (f, 0, 0)),
        ],
        out_specs=[
            pl.BlockSpec((1, BT, 1), lambda f, b: (f * NB + b, 0, 0)),
            pl.BlockSpec((1, K, D), lambda f, b: (f, 0, 0)),
        ],
        out_shape=[
            jax.ShapeDtypeStruct((F * NB, BT, 1), jnp.int32),
            jax.ShapeDtypeStruct((F, K, D), jnp.float32),
        ],
        compiler_params=pltpu.CompilerParams(
            dimension_semantics=("parallel", "arbitrary"),
        ),
        interpret=interpret,
    )(inputs, W)

    idx2d = idx.reshape(NW * NCH, ICH)
    wt2d = wt.reshape(F * K, D)
    if interpret:
        q = wt2d[idx2d.reshape(-1)]
        return q.reshape(F, B, D)

    gather = functools.partial(
        pl.kernel,
        mesh=plsc.VectorSubcoreMesh(core_axis_name="c", subcore_axis_name="s"),
        out_type=jax.ShapeDtypeStruct((F * B, D), jnp.float32),
        scratch_types=[
            pltpu.VMEM((NCH, ICH), jnp.int32),
            pltpu.VMEM((BPW, D), jnp.float32),
            pltpu.SemaphoreType.DMA,
        ],
        compiler_params=pltpu.CompilerParams(use_tc_tiling_on_sc=False),
    )(_gather_body)
    q = gather(wt2d, idx2d)
    return q.reshape(F, B, D)
